# baseline (speedup 1.0000x reference)
import functools

import jax
import jax.numpy as jnp
from jax import lax
from jax.experimental import pallas as pl
from jax.experimental.pallas import tpu as pltpu
from jax.experimental.pallas import tpu_sc as plsc

N = 10000
E = 160000
D = 256
H = 256
C = 4
CP = 16
NC = 2
NS = 16
KE = 100
KE5 = 125
KEP = 128
NCH_HALF = E // (NC * NS) // KE5
NCH_FULL = E // NS // KE
NPAD = 10240
RPT = NPAD // NS
BN = 1000
G = N // BN

_f32 = jnp.float32


_SC_PARAMS = pltpu.CompilerParams(use_tc_tiling_on_sc=False)


@functools.cache
def _mesh():
    return plsc.VectorSubcoreMesh(core_axis_name="c", subcore_axis_name="s",
                                  num_cores=NC, num_subcores=NS)


def _deg_body(dst_hbm, ones_hbm, zer_hbm, out_hbm, idx_v, ones_v, acc, sem):
    c = lax.axis_index("c")
    s = lax.axis_index("s")
    w = c * NS + s
    pltpu.sync_copy(zer_hbm, acc.at[pl.ds(s * RPT, RPT)])
    pltpu.sync_copy(ones_hbm, ones_v)
    pltpu.sync_copy(dst_hbm.at[w], idx_v)
    plsc.subcore_barrier()

    @pl.loop(0, NCH_HALF, step=8)
    def _(j):
        descs = [pltpu.async_copy(ones_v, acc.at[idx_v.at[j + b]], sem,
                                  add=True) for b in range(8)]
        for d in descs:
            d.wait()

    plsc.subcore_barrier()
    pltpu.sync_copy(acc.at[pl.ds(s * RPT, RPT)],
                    out_hbm.at[pl.ds(c * NPAD + s * RPT, RPT)])


@functools.cache
def _deg_call():
    return pl.kernel(
        _deg_body,
        out_type=jax.ShapeDtypeStruct((NC * NPAD, CP), _f32),
        mesh=_mesh(),
        scratch_types=[
            pltpu.VMEM((NCH_HALF, KE5), jnp.int32),
            pltpu.VMEM((KE5, CP), _f32),
            pltpu.VMEM_SHARED((NPAD, CP), _f32),
            pltpu.SemaphoreType.DMA,
        ],
        compiler_params=_SC_PARAMS,
    )


def _agg1_body(y_hbm, src_hbm, dst_hbm, zer_hbm, out_hbm, sidx, didx,
               b0, b1, acc, g0, g1, s0, s1):
    c = lax.axis_index("c")
    s = lax.axis_index("s")
    w = c * NS + s
    bufs = (b0, b1)
    gsem = (g0, g1)
    ssem = (s0, s1)
    pltpu.sync_copy(zer_hbm, acc.at[pl.ds(s * RPT, RPT)])
    pltpu.sync_copy(src_hbm.at[w], sidx)
    pltpu.sync_copy(dst_hbm.at[w], didx)
    plsc.subcore_barrier()

    def gather(jb, b, sem):
        return pltpu.async_copy(y_hbm.at[sidx.at[jb]], bufs[b], sem)

    def scatter(jb, b):
        return pltpu.async_copy(bufs[b], acc.at[didx.at[jb]], ssem[b],
                                add=True)

    for b in range(2):
        gather(b, b, gsem[b])

    @pl.loop(0, NCH_HALF - 2, step=2)
    def _(j):
        descs = []
        for b in range(2):
            pltpu.make_async_copy(y_hbm.at[sidx.at[j + b]], bufs[b],
                                  gsem[b]).wait()
            descs.append(scatter(j + b, b))
        for b in range(2):
            descs[b].wait()
            gather(j + b + 2, b, gsem[b])

    descs = []
    for b in range(2):
        jb = NCH_HALF - 2 + b
        pltpu.make_async_copy(y_hbm.at[sidx.at[jb]], bufs[b], gsem[b]).wait()
        descs.append(scatter(jb, b))
    for d in descs:
        d.wait()

    plsc.subcore_barrier()
    pltpu.sync_copy(acc.at[pl.ds(s * RPT, RPT)],
                    out_hbm.at[pl.ds(c * NPAD + s * RPT, RPT)])


@functools.cache
def _agg1_call():
    return pl.kernel(
        _agg1_body,
        out_type=jax.ShapeDtypeStruct((NC * NPAD, 2, 128), jnp.bfloat16),
        mesh=_mesh(),
        scratch_types=[
            pltpu.VMEM((NCH_HALF, KE5), jnp.int32),
            pltpu.VMEM((NCH_HALF, KE5), jnp.int32),
            pltpu.VMEM((KE5, 2, 128), jnp.bfloat16),
            pltpu.VMEM((KE5, 2, 128), jnp.bfloat16),
            pltpu.VMEM_SHARED((NPAD, 2, 128), jnp.bfloat16),
        ] + [pltpu.SemaphoreType.DMA] * 4,
        compiler_params=_SC_PARAMS,
    )


def _agg2_body(z_hbm, src_hbm, dst_hbm, zer_hbm, out_hbm, sidx, didx,
               b0, b1, b2, b3, acc, g0, g1, g2, g3, s0, s1, s2, s3):
    c = lax.axis_index("c")
    s = lax.axis_index("s")
    w = c * NS + s
    bufs = (b0, b1, b2, b3)
    gsem = (g0, g1, g2, g3)
    ssem = (s0, s1, s2, s3)
    pltpu.sync_copy(zer_hbm, acc.at[pl.ds(s * RPT, RPT)])
    pltpu.sync_copy(src_hbm.at[w], sidx)
    pltpu.sync_copy(dst_hbm.at[w], didx)
    plsc.subcore_barrier()

    def gather(jb, b, sem):
        return pltpu.async_copy(z_hbm.at[sidx.at[jb]], bufs[b], sem)

    def scatter(jb, b):
        return pltpu.async_copy(bufs[b], acc.at[didx.at[jb]], ssem[b],
                                add=True)

    for b in range(4):
        gather(b, b, gsem[b])

    @pl.loop(0, NCH_HALF - 4, step=4)
    def _(j):
        descs = []
        for b in range(4):
            pltpu.make_async_copy(z_hbm.at[sidx.at[j + b]], bufs[b],
                                  gsem[b]).wait()
            descs.append(scatter(j + b, b))
        for b in range(4):
            descs[b].wait()
            gather(j + b + 4, b, gsem[b])

    descs = []
    for b in range(4):
        jb = NCH_HALF - 4 + b
        pltpu.make_async_copy(z_hbm.at[sidx.at[jb]], bufs[b], gsem[b]).wait()
        descs.append(scatter(jb, b))
    for d in descs:
        d.wait()

    plsc.subcore_barrier()
    pltpu.sync_copy(acc.at[pl.ds(s * RPT, RPT)],
                    out_hbm.at[pl.ds(c * NPAD + s * RPT, RPT)])


@functools.cache
def _agg2_call():
    return pl.kernel(
        _agg2_body,
        out_type=jax.ShapeDtypeStruct((NC * NPAD, CP), _f32),
        mesh=_mesh(),
        scratch_types=[
            pltpu.VMEM((NCH_HALF, KE5), jnp.int32),
            pltpu.VMEM((NCH_HALF, KE5), jnp.int32),
            pltpu.VMEM((KE5, CP), _f32),
            pltpu.VMEM((KE5, CP), _f32),
            pltpu.VMEM((KE5, CP), _f32),
            pltpu.VMEM((KE5, CP), _f32),
            pltpu.VMEM_SHARED((NPAD, CP), _f32),
        ] + [pltpu.SemaphoreType.DMA] * 8,
        compiler_params=_SC_PARAMS,
    )


def _dinv_from(dega):
    return jax.lax.rsqrt(dega[0, :, 0] + dega[1, :, 0] + 1.0)[:, None]


def _y_body(dega_ref, x_ref, w1_ref, yb_ref):
    dinv = _dinv_from(dega_ref[...])
    y = jnp.dot(x_ref[...], w1_ref[...], preferred_element_type=_f32) * dinv
    yb_ref[...] = y.astype(jnp.bfloat16).reshape(BN, 2, 128)


_y_call = pl.pallas_call(
    _y_body,
    grid=(G,),
    in_specs=[
        pl.BlockSpec((2, BN, CP), lambda i: (0, i, 0)),
        pl.BlockSpec((BN, D), lambda i: (i, 0)),
        pl.BlockSpec((D, H), lambda i: (0, 0)),
    ],
    out_specs=pl.BlockSpec((BN, 2, 128), lambda i: (i, 0, 0)),
    out_shape=jax.ShapeDtypeStruct((N, 2, 128), jnp.bfloat16),
)


def _ev_od(packed_f32):
    u = jax.lax.bitcast_convert_type(packed_f32, jnp.uint32)
    ev = jax.lax.bitcast_convert_type(u << jnp.uint32(16), _f32)
    od = jax.lax.bitcast_convert_type(u & jnp.uint32(0xFFFF0000), _f32)
    return ev, od


def _h_body(dega_ref, s1_ref, yb_ref, w2_ref, b1_ref, z_ref):
    dinv = _dinv_from(dega_ref[...])
    e0, o0 = _ev_od(s1_ref[0])
    e1, o1 = _ev_od(s1_ref[1])
    ey, oy = _ev_od(yb_ref[...])
    t = jnp.concatenate([e0 + e1 + ey, o0 + o1 + oy], axis=1)
    h = jnp.maximum(t * dinv + b1_ref[...], 0.0)
    z_ref[...] = jnp.dot(h, w2_ref[...], preferred_element_type=_f32) * dinv


_h_call = pl.pallas_call(
    _h_body,
    grid=(G,),
    in_specs=[
        pl.BlockSpec((2, BN, CP), lambda i: (0, i, 0)),
        pl.BlockSpec((2, BN, 128), lambda i: (0, i, 0)),
        pl.BlockSpec((BN, 128), lambda i: (i, 0)),
        pl.BlockSpec((H, CP), lambda i: (0, 0)),
        pl.BlockSpec((1, H), lambda i: (0, 0)),
    ],
    out_specs=pl.BlockSpec((BN, CP), lambda i: (i, 0)),
    out_shape=jax.ShapeDtypeStruct((N, CP), _f32),
)


def _out_body(dega_ref, s2_ref, z_ref, b2_ref, o_ref):
    dinv = _dinv_from(dega_ref[...])
    o = (s2_ref[0] + s2_ref[1] + z_ref[...]) * dinv + b2_ref[...]
    col = jax.lax.broadcasted_iota(jnp.int32, o.shape, 1)
    mask = col < C
    m = jnp.max(jnp.where(mask, o, _f32(-1e30)), axis=1, keepdims=True)
    e = jnp.where(mask, jnp.exp(o - m), 0.0)
    lse = jnp.log(jnp.sum(e, axis=1, keepdims=True))
    o_ref[...] = o - m - lse


_out_call = pl.pallas_call(
    _out_body,
    grid=(G,),
    in_specs=[
        pl.BlockSpec((2, BN, CP), lambda i: (0, i, 0)),
        pl.BlockSpec((2, BN, CP), lambda i: (0, i, 0)),
        pl.BlockSpec((BN, CP), lambda i: (i, 0)),
        pl.BlockSpec((1, CP), lambda i: (0, 0)),
    ],
    out_specs=pl.BlockSpec((BN, CP), lambda i: (i, 0)),
    out_shape=jax.ShapeDtypeStruct((N, CP), _f32),
)


def kernel(x, edge_index, W1, b1, W2, b2):
    src = edge_index[0]
    dst = edge_index[1]

    def _lin(flat, shape):
        r = flat.size // 128
        return lax.optimization_barrier(flat.reshape(r, 128)).reshape(shape)

    dst_half = _lin(dst, (NC * NS, NCH_HALF, KE5))
    src_half = _lin(src, (NC * NS, NCH_HALF, KE5))


    ones16 = jnp.ones((KE5, CP), _f32)
    zer16 = jnp.zeros((RPT, CP), _f32)
    zer256 = jnp.zeros((RPT, 2, 128), jnp.bfloat16)
    q = jnp.concatenate([jnp.arange(0, H, 2), jnp.arange(1, H, 2)])
    w2p = jnp.pad(W2, ((0, 0), (0, CP - C)))[q]
    b1r = b1[q].reshape(1, H)
    b2p = jnp.pad(b2, (0, CP - C)).reshape(1, CP)

    def _f32view(a, shape):
        b = jax.lax.bitcast_convert_type(
            a.reshape(a.shape[:-2] + (2, 64, 2)), _f32)
        return b.reshape(shape)

    dega = _deg_call()(dst_half, ones16, zer16).reshape(NC, NPAD, CP)
    yb = _y_call(dega, x, W1)
    s1 = _agg1_call()(yb, src_half, dst_half, zer256)
    z = _h_call(dega, _f32view(s1, (NC, NPAD, 128)),
                _f32view(yb, (N, 128)), w2p, b1r)
    s2 = _agg2_call()(z, src_half, dst_half, zer16).reshape(NC, NPAD, CP)
    out = _out_call(dega, s2, z, b2p)
    return out[:, :C]


# speedup vs baseline: 1.5798x; 1.5798x over previous
import functools

import jax
import jax.numpy as jnp
from jax import lax
from jax.experimental import pallas as pl
from jax.experimental.pallas import tpu as pltpu
from jax.experimental.pallas import tpu_sc as plsc

N = 10000
E = 160000
D = 256
H = 256
C = 4
CP = 16
NC = 2
NS = 16
KE = 100
KE5 = 125
KEP = 128
NCH_HALF = E // (NC * NS) // KE5
NCH_FULL = E // NS // KE
NPAD = 10240
RPT = NPAD // NS
BN = 1000
G = N // BN

_f32 = jnp.float32


_SC_PARAMS = pltpu.CompilerParams(use_tc_tiling_on_sc=False)


@functools.cache
def _mesh():
    return plsc.VectorSubcoreMesh(core_axis_name="c", subcore_axis_name="s",
                                  num_cores=NC, num_subcores=NS)


def _deg_body(dst_hbm, ones_hbm, zer_hbm, out_hbm, idx_v, ones_v, acc, sem):
    c = lax.axis_index("c")
    s = lax.axis_index("s")
    w = c * NS + s
    pltpu.sync_copy(zer_hbm, acc.at[pl.ds(s * RPT, RPT)])
    pltpu.sync_copy(ones_hbm, ones_v)
    pltpu.sync_copy(dst_hbm.at[w], idx_v)
    plsc.subcore_barrier()

    @pl.loop(0, NCH_HALF, step=8)
    def _(j):
        descs = [pltpu.async_copy(ones_v, acc.at[idx_v.at[j + b]], sem,
                                  add=True) for b in range(8)]
        for d in descs:
            d.wait()

    plsc.subcore_barrier()
    pltpu.sync_copy(acc.at[pl.ds(s * RPT, RPT)],
                    out_hbm.at[pl.ds(c * NPAD + s * RPT, RPT)])


@functools.cache
def _deg_call():
    return pl.kernel(
        _deg_body,
        out_type=jax.ShapeDtypeStruct((NC * NPAD, CP), _f32),
        mesh=_mesh(),
        scratch_types=[
            pltpu.VMEM((NCH_HALF, KE5), jnp.int32),
            pltpu.VMEM((KE5, CP), _f32),
            pltpu.VMEM_SHARED((NPAD, CP), _f32),
            pltpu.SemaphoreType.DMA,
        ],
        compiler_params=_SC_PARAMS,
    )


def _agg1_body(y_hbm, src_hbm, dst_hbm, zer_hbm, out_hbm, sidx, didx,
               b0, b1, acc, g0, g1, s0, s1):
    c = lax.axis_index("c")
    s = lax.axis_index("s")
    w = c * NS + s
    bufs = (b0, b1)
    gsem = (g0, g1)
    ssem = (s0, s1)
    pltpu.sync_copy(zer_hbm, acc.at[pl.ds(s * RPT, RPT)])
    pltpu.sync_copy(src_hbm.at[w], sidx)
    pltpu.sync_copy(dst_hbm.at[w], didx)
    plsc.subcore_barrier()

    def gather(jb, b, sem):
        return pltpu.async_copy(y_hbm.at[sidx.at[jb]], bufs[b], sem)

    def scatter(jb, b):
        return pltpu.async_copy(bufs[b], acc.at[didx.at[jb]], ssem[b],
                                add=True)

    for b in range(2):
        gather(b, b, gsem[b])

    @pl.loop(0, NCH_HALF - 2, step=2)
    def _(j):
        descs = []
        for b in range(2):
            pltpu.make_async_copy(y_hbm.at[sidx.at[j + b]], bufs[b],
                                  gsem[b]).wait()
            descs.append(scatter(j + b, b))
        for b in range(2):
            descs[b].wait()
            gather(j + b + 2, b, gsem[b])

    descs = []
    for b in range(2):
        jb = NCH_HALF - 2 + b
        pltpu.make_async_copy(y_hbm.at[sidx.at[jb]], bufs[b], gsem[b]).wait()
        descs.append(scatter(jb, b))
    for d in descs:
        d.wait()

    plsc.subcore_barrier()
    pltpu.sync_copy(acc.at[pl.ds(s * RPT, RPT)],
                    out_hbm.at[pl.ds(c * NPAD + s * RPT, RPT)])


@functools.cache
def _agg1_call():
    return pl.kernel(
        _agg1_body,
        out_type=jax.ShapeDtypeStruct((NC * NPAD, 2, 128), jnp.bfloat16),
        mesh=_mesh(),
        scratch_types=[
            pltpu.VMEM((NCH_HALF, KE5), jnp.int32),
            pltpu.VMEM((NCH_HALF, KE5), jnp.int32),
            pltpu.VMEM((KE5, 2, 128), jnp.bfloat16),
            pltpu.VMEM((KE5, 2, 128), jnp.bfloat16),
            pltpu.VMEM_SHARED((NPAD, 2, 128), jnp.bfloat16),
        ] + [pltpu.SemaphoreType.DMA] * 4,
        compiler_params=_SC_PARAMS,
    )


def _agg2_body(z_hbm, src_hbm, dst_hbm, zer_hbm, out_hbm, sidx, didx,
               b0, b1, b2, b3, acc, g0, g1, g2, g3, s0, s1, s2, s3):
    c = lax.axis_index("c")
    s = lax.axis_index("s")
    w = c * NS + s
    bufs = (b0, b1, b2, b3)
    gsem = (g0, g1, g2, g3)
    ssem = (s0, s1, s2, s3)
    pltpu.sync_copy(zer_hbm, acc.at[pl.ds(s * RPT, RPT)])
    pltpu.sync_copy(src_hbm.at[w], sidx)
    pltpu.sync_copy(dst_hbm.at[w], didx)
    plsc.subcore_barrier()

    def gather(jb, b, sem):
        return pltpu.async_copy(z_hbm.at[sidx.at[jb]], bufs[b], sem)

    def scatter(jb, b):
        return pltpu.async_copy(bufs[b], acc.at[didx.at[jb]], ssem[b],
                                add=True)

    for b in range(4):
        gather(b, b, gsem[b])

    @pl.loop(0, NCH_HALF - 4, step=4)
    def _(j):
        descs = []
        for b in range(4):
            pltpu.make_async_copy(z_hbm.at[sidx.at[j + b]], bufs[b],
                                  gsem[b]).wait()
            descs.append(scatter(j + b, b))
        for b in range(4):
            descs[b].wait()
            gather(j + b + 4, b, gsem[b])

    descs = []
    for b in range(4):
        jb = NCH_HALF - 4 + b
        pltpu.make_async_copy(z_hbm.at[sidx.at[jb]], bufs[b], gsem[b]).wait()
        descs.append(scatter(jb, b))
    for d in descs:
        d.wait()

    plsc.subcore_barrier()
    pltpu.sync_copy(acc.at[pl.ds(s * RPT, RPT)],
                    out_hbm.at[pl.ds(c * NPAD + s * RPT, RPT)])


@functools.cache
def _agg2_call():
    return pl.kernel(
        _agg2_body,
        out_type=jax.ShapeDtypeStruct((NC * NPAD, CP), _f32),
        mesh=_mesh(),
        scratch_types=[
            pltpu.VMEM((NCH_HALF, KE5), jnp.int32),
            pltpu.VMEM((NCH_HALF, KE5), jnp.int32),
            pltpu.VMEM((KE5, CP), _f32),
            pltpu.VMEM((KE5, CP), _f32),
            pltpu.VMEM((KE5, CP), _f32),
            pltpu.VMEM((KE5, CP), _f32),
            pltpu.VMEM_SHARED((NPAD, CP), _f32),
        ] + [pltpu.SemaphoreType.DMA] * 8,
        compiler_params=_SC_PARAMS,
    )


def _dinv_from(dega):
    return jax.lax.rsqrt(dega[0, :, 0] + dega[1, :, 0] + 1.0)[:, None]


def _y_body(dega_ref, x_ref, w1_ref, y_ref, yb_ref):
    dinv = _dinv_from(dega_ref[...])
    y = jnp.dot(x_ref[...], w1_ref[...], preferred_element_type=_f32) * dinv
    y_ref[0] = y[:, :128]
    y_ref[1] = y[:, 128:]
    yb_ref[...] = y.astype(jnp.bfloat16).reshape(BN, 2, 128)


_y_call = pl.pallas_call(
    _y_body,
    grid=(G,),
    in_specs=[
        pl.BlockSpec((2, BN, CP), lambda i: (0, i, 0)),
        pl.BlockSpec((BN, D), lambda i: (i, 0)),
        pl.BlockSpec((D, H), lambda i: (0, 0)),
    ],
    out_specs=[pl.BlockSpec((2, BN, 128), lambda i: (0, i, 0)),
               pl.BlockSpec((BN, 2, 128), lambda i: (i, 0, 0))],
    out_shape=[jax.ShapeDtypeStruct((2, N, 128), _f32),
               jax.ShapeDtypeStruct((N, 2, 128), jnp.bfloat16)],
)


def _h_body(dega_ref, s1_ref, y2_ref, w2_ref, b1_ref, z_ref):
    dinv = _dinv_from(dega_ref[...])
    s1 = s1_ref[...].astype(_f32)
    t = (s1[0] + s1[1]
         + jnp.concatenate([y2_ref[0], y2_ref[1]], axis=1))
    h = jnp.maximum(t * dinv + b1_ref[...], 0.0)
    z_ref[...] = jnp.dot(h, w2_ref[...], preferred_element_type=_f32) * dinv


_h_call = pl.pallas_call(
    _h_body,
    grid=(G,),
    in_specs=[
        pl.BlockSpec((2, BN, CP), lambda i: (0, i, 0)),
        pl.BlockSpec((2, BN, H), lambda i: (0, i, 0)),
        pl.BlockSpec((2, BN, 128), lambda i: (0, i, 0)),
        pl.BlockSpec((H, CP), lambda i: (0, 0)),
        pl.BlockSpec((1, H), lambda i: (0, 0)),
    ],
    out_specs=pl.BlockSpec((BN, CP), lambda i: (i, 0)),
    out_shape=jax.ShapeDtypeStruct((N, CP), _f32),
)


def _out_body(dega_ref, s2_ref, z_ref, b2_ref, o_ref):
    dinv = _dinv_from(dega_ref[...])
    o = (s2_ref[0] + s2_ref[1] + z_ref[...]) * dinv + b2_ref[...]
    col = jax.lax.broadcasted_iota(jnp.int32, o.shape, 1)
    mask = col < C
    m = jnp.max(jnp.where(mask, o, _f32(-1e30)), axis=1, keepdims=True)
    e = jnp.where(mask, jnp.exp(o - m), 0.0)
    lse = jnp.log(jnp.sum(e, axis=1, keepdims=True))
    o_ref[...] = o - m - lse


_out_call = pl.pallas_call(
    _out_body,
    grid=(G,),
    in_specs=[
        pl.BlockSpec((2, BN, CP), lambda i: (0, i, 0)),
        pl.BlockSpec((2, BN, CP), lambda i: (0, i, 0)),
        pl.BlockSpec((BN, CP), lambda i: (i, 0)),
        pl.BlockSpec((1, CP), lambda i: (0, 0)),
    ],
    out_specs=pl.BlockSpec((BN, CP), lambda i: (i, 0)),
    out_shape=jax.ShapeDtypeStruct((N, CP), _f32),
)


def kernel(x, edge_index, W1, b1, W2, b2):
    src = edge_index[0]
    dst = edge_index[1]

    def _lin(flat, shape):
        r = flat.size // 128
        return lax.optimization_barrier(flat.reshape(r, 128)).reshape(shape)

    dst_half = _lin(dst, (NC * NS, NCH_HALF, KE5))
    src_half = _lin(src, (NC * NS, NCH_HALF, KE5))


    ones16 = jnp.ones((KE5, CP), _f32)
    zer16 = jnp.zeros((RPT, CP), _f32)
    zer256 = jnp.zeros((RPT, 2, 128), jnp.bfloat16)
    w2p = jnp.pad(W2, ((0, 0), (0, CP - C)))
    b1r = b1.reshape(1, H)
    b2p = jnp.pad(b2, (0, CP - C)).reshape(1, CP)

    dega = _deg_call()(dst_half, ones16, zer16).reshape(NC, NPAD, CP)
    y2, yb = _y_call(dega, x, W1)
    s1 = _agg1_call()(yb, src_half, dst_half, zer256).reshape(NC, NPAD, H)
    z = _h_call(dega, s1, y2, w2p, b1r)
    s2 = _agg2_call()(z, src_half, dst_half, zer16).reshape(NC, NPAD, CP)
    out = _out_call(dega, s2, z, b2p)
    return out[:, :C]


# speedup vs baseline: 1.6154x; 1.0225x over previous
import functools

import jax
import jax.numpy as jnp
from jax import lax
from jax.experimental import pallas as pl
from jax.experimental.pallas import tpu as pltpu
from jax.experimental.pallas import tpu_sc as plsc

N = 10000
E = 160000
D = 256
H = 256
C = 4
CP = 16
NC = 2
NS = 16
KE = 100
KE5 = 125
KEP = 128
NCH_HALF = E // (NC * NS) // KE5
NCH_FULL = E // NS // KE
NPAD = 10240
RPT = NPAD // NS
BN = 2000
G = N // BN

_f32 = jnp.float32


_SC_PARAMS = pltpu.CompilerParams(use_tc_tiling_on_sc=False)


@functools.cache
def _mesh():
    return plsc.VectorSubcoreMesh(core_axis_name="c", subcore_axis_name="s",
                                  num_cores=NC, num_subcores=NS)


def _deg_body(dst_hbm, ones_hbm, zer_hbm, out_hbm, idx_v, ones_v, acc, sem):
    c = lax.axis_index("c")
    s = lax.axis_index("s")
    w = c * NS + s
    pltpu.sync_copy(zer_hbm, acc.at[pl.ds(s * RPT, RPT)])
    pltpu.sync_copy(ones_hbm, ones_v)
    pltpu.sync_copy(dst_hbm.at[w], idx_v)
    plsc.subcore_barrier()

    @pl.loop(0, NCH_HALF, step=8)
    def _(j):
        descs = [pltpu.async_copy(ones_v, acc.at[idx_v.at[j + b]], sem,
                                  add=True) for b in range(8)]
        for d in descs:
            d.wait()

    plsc.subcore_barrier()
    pltpu.sync_copy(acc.at[pl.ds(s * RPT, RPT)],
                    out_hbm.at[pl.ds(c * NPAD + s * RPT, RPT)])


@functools.cache
def _deg_call():
    return pl.kernel(
        _deg_body,
        out_type=jax.ShapeDtypeStruct((NC * NPAD, CP), _f32),
        mesh=_mesh(),
        scratch_types=[
            pltpu.VMEM((NCH_HALF, KE5), jnp.int32),
            pltpu.VMEM((KE5, CP), _f32),
            pltpu.VMEM_SHARED((NPAD, CP), _f32),
            pltpu.SemaphoreType.DMA,
        ],
        compiler_params=_SC_PARAMS,
    )


def _agg1_body(y_hbm, src_hbm, dst_hbm, zer_hbm, out_hbm, sidx, didx,
               b0, b1, acc, g0, g1, s0, s1):
    c = lax.axis_index("c")
    s = lax.axis_index("s")
    w = c * NS + s
    bufs = (b0, b1)
    gsem = (g0, g1)
    ssem = (s0, s1)
    pltpu.sync_copy(zer_hbm, acc.at[pl.ds(s * RPT, RPT)])
    pltpu.sync_copy(src_hbm.at[w], sidx)
    pltpu.sync_copy(dst_hbm.at[w], didx)
    plsc.subcore_barrier()

    def gather(jb, b, sem):
        return pltpu.async_copy(y_hbm.at[sidx.at[jb]], bufs[b], sem)

    def scatter(jb, b):
        return pltpu.async_copy(bufs[b], acc.at[didx.at[jb]], ssem[b],
                                add=True)

    for b in range(2):
        gather(b, b, gsem[b])

    @pl.loop(0, NCH_HALF - 2, step=2)
    def _(j):
        descs = []
        for b in range(2):
            pltpu.make_async_copy(y_hbm.at[sidx.at[j + b]], bufs[b],
                                  gsem[b]).wait()
            descs.append(scatter(j + b, b))
        for b in range(2):
            descs[b].wait()
            gather(j + b + 2, b, gsem[b])

    descs = []
    for b in range(2):
        jb = NCH_HALF - 2 + b
        pltpu.make_async_copy(y_hbm.at[sidx.at[jb]], bufs[b], gsem[b]).wait()
        descs.append(scatter(jb, b))
    for d in descs:
        d.wait()

    plsc.subcore_barrier()
    pltpu.sync_copy(acc.at[pl.ds(s * RPT, RPT)],
                    out_hbm.at[pl.ds(c * NPAD + s * RPT, RPT)])


@functools.cache
def _agg1_call():
    return pl.kernel(
        _agg1_body,
        out_type=jax.ShapeDtypeStruct((NC * NPAD, 2, 128), jnp.bfloat16),
        mesh=_mesh(),
        scratch_types=[
            pltpu.VMEM((NCH_HALF, KE5), jnp.int32),
            pltpu.VMEM((NCH_HALF, KE5), jnp.int32),
            pltpu.VMEM((KE5, 2, 128), jnp.bfloat16),
            pltpu.VMEM((KE5, 2, 128), jnp.bfloat16),
            pltpu.VMEM_SHARED((NPAD, 2, 128), jnp.bfloat16),
        ] + [pltpu.SemaphoreType.DMA] * 4,
        compiler_params=_SC_PARAMS,
    )


def _agg2_body(z_hbm, src_hbm, dst_hbm, zer_hbm, out_hbm, sidx, didx,
               b0, b1, b2, b3, acc, g0, g1, g2, g3, s0, s1, s2, s3):
    c = lax.axis_index("c")
    s = lax.axis_index("s")
    w = c * NS + s
    bufs = (b0, b1, b2, b3)
    gsem = (g0, g1, g2, g3)
    ssem = (s0, s1, s2, s3)
    pltpu.sync_copy(zer_hbm, acc.at[pl.ds(s * RPT, RPT)])
    pltpu.sync_copy(src_hbm.at[w], sidx)
    pltpu.sync_copy(dst_hbm.at[w], didx)
    plsc.subcore_barrier()

    def gather(jb, b, sem):
        return pltpu.async_copy(z_hbm.at[sidx.at[jb]], bufs[b], sem)

    def scatter(jb, b):
        return pltpu.async_copy(bufs[b], acc.at[didx.at[jb]], ssem[b],
                                add=True)

    for b in range(4):
        gather(b, b, gsem[b])

    @pl.loop(0, NCH_HALF - 4, step=4)
    def _(j):
        descs = []
        for b in range(4):
            pltpu.make_async_copy(z_hbm.at[sidx.at[j + b]], bufs[b],
                                  gsem[b]).wait()
            descs.append(scatter(j + b, b))
        for b in range(4):
            descs[b].wait()
            gather(j + b + 4, b, gsem[b])

    descs = []
    for b in range(4):
        jb = NCH_HALF - 4 + b
        pltpu.make_async_copy(z_hbm.at[sidx.at[jb]], bufs[b], gsem[b]).wait()
        descs.append(scatter(jb, b))
    for d in descs:
        d.wait()

    plsc.subcore_barrier()
    pltpu.sync_copy(acc.at[pl.ds(s * RPT, RPT)],
                    out_hbm.at[pl.ds(c * NPAD + s * RPT, RPT)])


@functools.cache
def _agg2_call():
    return pl.kernel(
        _agg2_body,
        out_type=jax.ShapeDtypeStruct((NC * NPAD, CP), _f32),
        mesh=_mesh(),
        scratch_types=[
            pltpu.VMEM((NCH_HALF, KE5), jnp.int32),
            pltpu.VMEM((NCH_HALF, KE5), jnp.int32),
            pltpu.VMEM((KE5, CP), _f32),
            pltpu.VMEM((KE5, CP), _f32),
            pltpu.VMEM((KE5, CP), _f32),
            pltpu.VMEM((KE5, CP), _f32),
            pltpu.VMEM_SHARED((NPAD, CP), _f32),
        ] + [pltpu.SemaphoreType.DMA] * 8,
        compiler_params=_SC_PARAMS,
    )


def _dinv_from(dega):
    return jax.lax.rsqrt(dega[0, :, 0] + dega[1, :, 0] + 1.0)[:, None]


def _y_body(dega_ref, x_ref, w1_ref, y_ref, yb_ref):
    dinv = _dinv_from(dega_ref[...])
    y = jnp.dot(x_ref[...], w1_ref[...], preferred_element_type=_f32) * dinv
    y_ref[0] = y[:, :128]
    y_ref[1] = y[:, 128:]
    yb_ref[...] = y.astype(jnp.bfloat16).reshape(BN, 2, 128)


_y_call = pl.pallas_call(
    _y_body,
    grid=(G,),
    in_specs=[
        pl.BlockSpec((2, BN, CP), lambda i: (0, i, 0)),
        pl.BlockSpec((BN, D), lambda i: (i, 0)),
        pl.BlockSpec((D, H), lambda i: (0, 0)),
    ],
    out_specs=[pl.BlockSpec((2, BN, 128), lambda i: (0, i, 0)),
               pl.BlockSpec((BN, 2, 128), lambda i: (i, 0, 0))],
    out_shape=[jax.ShapeDtypeStruct((2, N, 128), _f32),
               jax.ShapeDtypeStruct((N, 2, 128), jnp.bfloat16)],
)


def _h_body(dega_ref, s1_ref, y2_ref, w2_ref, b1_ref, z_ref):
    dinv = _dinv_from(dega_ref[...])
    s1 = s1_ref[...].astype(_f32)
    t = (s1[0] + s1[1]
         + jnp.concatenate([y2_ref[0], y2_ref[1]], axis=1))
    h = jnp.maximum(t * dinv + b1_ref[...], 0.0)
    z_ref[...] = jnp.dot(h, w2_ref[...], preferred_element_type=_f32) * dinv


_h_call = pl.pallas_call(
    _h_body,
    grid=(G,),
    in_specs=[
        pl.BlockSpec((2, BN, CP), lambda i: (0, i, 0)),
        pl.BlockSpec((2, BN, H), lambda i: (0, i, 0)),
        pl.BlockSpec((2, BN, 128), lambda i: (0, i, 0)),
        pl.BlockSpec((H, CP), lambda i: (0, 0)),
        pl.BlockSpec((1, H), lambda i: (0, 0)),
    ],
    out_specs=pl.BlockSpec((BN, CP), lambda i: (i, 0)),
    out_shape=jax.ShapeDtypeStruct((N, CP), _f32),
)


def _out_body(dega_ref, s2_ref, z_ref, b2_ref, o_ref):
    dinv = _dinv_from(dega_ref[...])
    o = (s2_ref[0] + s2_ref[1] + z_ref[...]) * dinv + b2_ref[...]
    col = jax.lax.broadcasted_iota(jnp.int32, o.shape, 1)
    mask = col < C
    m = jnp.max(jnp.where(mask, o, _f32(-1e30)), axis=1, keepdims=True)
    e = jnp.where(mask, jnp.exp(o - m), 0.0)
    lse = jnp.log(jnp.sum(e, axis=1, keepdims=True))
    o_ref[...] = o - m - lse


_out_call = pl.pallas_call(
    _out_body,
    grid=(G,),
    in_specs=[
        pl.BlockSpec((2, BN, CP), lambda i: (0, i, 0)),
        pl.BlockSpec((2, BN, CP), lambda i: (0, i, 0)),
        pl.BlockSpec((BN, CP), lambda i: (i, 0)),
        pl.BlockSpec((1, CP), lambda i: (0, 0)),
    ],
    out_specs=pl.BlockSpec((BN, CP), lambda i: (i, 0)),
    out_shape=jax.ShapeDtypeStruct((N, CP), _f32),
)


def kernel(x, edge_index, W1, b1, W2, b2):
    src = edge_index[0]
    dst = edge_index[1]

    def _lin(flat, shape):
        r = flat.size // 128
        return lax.optimization_barrier(flat.reshape(r, 128)).reshape(shape)

    dst_half = _lin(dst, (NC * NS, NCH_HALF, KE5))
    src_half = _lin(src, (NC * NS, NCH_HALF, KE5))


    ones16 = jnp.ones((KE5, CP), _f32)
    zer16 = jnp.zeros((RPT, CP), _f32)
    zer256 = jnp.zeros((RPT, 2, 128), jnp.bfloat16)
    w2p = jnp.pad(W2, ((0, 0), (0, CP - C)))
    b1r = b1.reshape(1, H)
    b2p = jnp.pad(b2, (0, CP - C)).reshape(1, CP)

    dega = _deg_call()(dst_half, ones16, zer16).reshape(NC, NPAD, CP)
    y2, yb = _y_call(dega, x, W1)
    s1 = _agg1_call()(yb, src_half, dst_half, zer256).reshape(NC, NPAD, H)
    z = _h_call(dega, s1, y2, w2p, b1r)
    s2 = _agg2_call()(z, src_half, dst_half, zer16).reshape(NC, NPAD, CP)
    out = _out_call(dega, s2, z, b2p)
    return out[:, :C]
